# baseline (speedup 1.0000x reference)
import jax
import jax.numpy as jnp
from jax import lax
from jax.experimental import pallas as pl
from jax.experimental.pallas import tpu as pltpu
from jax.experimental.pallas import tpu_sc as plsc

_B = 4096
_L = 200
_D = 50
_DP = 64
_W32 = _DP // 2
_OUT = 20
_V = 100000

_NC = 2
_NS = 16
_NW = _NC * _NS
_SPT = _B // _NW

_CH = 32
_NCH = (_L + _CH - 1) // _CH
_LP = _NCH * _CH

_ROWS_BLK = 200
_PACK_GRID = (_V // 4) // _ROWS_BLK

_HI = 0xFFFF0000 - (1 << 32)


def _rne(b):
    lsb = lax.bitwise_and(lax.shift_right_logical(b, 16), jnp.int32(1))
    return lax.add(b, lax.add(jnp.int32(0x7FFF), lsb))


def _pack_words(y):
    y = jnp.pad(y, ((0, 0), (0, _DP - _D)))
    bits = lax.bitcast_convert_type(y, jnp.int32)
    lo = _rne(bits[:, :_W32])
    hi = _rne(bits[:, _W32:])
    return lax.bitwise_or(lax.bitwise_and(hi, jnp.int32(_HI)),
                          lax.shift_right_logical(lo, 16))


def _pack_body(t0, t1, t2, t3, o_ref):
    for q, t in enumerate((t0, t1, t2, t3)):
        o_ref[:, q * _W32:(q + 1) * _W32] = _pack_words(t[...])


def _pack(table):
    qb = (_V // 4) // _ROWS_BLK
    return pl.pallas_call(
        _pack_body,
        grid=(_PACK_GRID,),
        in_specs=[pl.BlockSpec((_ROWS_BLK, _D),
                               lambda i, q=q: (i + q * qb, 0))
                  for q in range(4)],
        out_specs=pl.BlockSpec((_ROWS_BLK, 128), lambda i: (i, 0)),
        out_shape=jax.ShapeDtypeStruct((_V // 4, 128), jnp.int32),
    )(table, table, table, table)


def _embed_sums(x, lengths, packed):
    mesh = plsc.VectorSubcoreMesh(core_axis_name="c", subcore_axis_name="s")

    def body(x_hbm, len_hbm, tbl, sums_hbm, xv, lv, acc, buf0, buf1,
             sem0, sem1):
        wid = lax.axis_index("s") * _NC + lax.axis_index("c")
        base = wid * _SPT
        pltpu.sync_copy(x_hbm.at[pl.ds(base, _SPT)], xv)
        pltpu.sync_copy(len_hbm.at[pl.ds(base, _SPT)], lv.at[pl.ds(0, _SPT)])

        def nchunks(s):
            n = lv[pl.ds(s, 16)][0]
            return (n + (_CH - 1)) // _CH

        def fire(s, buf, sem):
            m = nchunks(s)

            @pl.loop(0, m)
            def _(c):
                pltpu.async_copy(
                    tbl.at[xv.at[s, pl.ds(c * _CH, _CH)]],
                    buf.at[pl.ds(c * _CH, _CH)], sem)

        def drain(s, buf, sem):
            m = nchunks(s)

            @pl.loop(0, m)
            def _(c):
                pltpu.make_async_copy(
                    tbl.at[xv.at[s, pl.ds(c * _CH, _CH)]],
                    buf.at[pl.ds(c * _CH, _CH)], sem).wait()

        def halves(buf, j, col):
            w = buf[j, pl.ds(col, 16)]
            lo = plsc.bitcast(lax.shift_left(w, 16), jnp.float32)
            hi = plsc.bitcast(lax.bitwise_and(w, jnp.int32(_HI)), jnp.float32)
            return lo, hi

        def process(s, buf):
            n = lv[pl.ds(s, 16)][0]
            z = jnp.zeros((16,), jnp.float32)
            n2 = (n // 2) * 2

            def add_row(j, a0, a1, a2, a3):
                lo0, hi0 = halves(buf, j, 0)
                lo1, hi1 = halves(buf, j, 16)
                return a0 + lo0, a1 + lo1, a2 + hi0, a3 + hi1

            def accum2(j, carry):
                a0, a1, a2, a3, b0, b1, b2, b3 = carry
                a0, a1, a2, a3 = add_row(j, a0, a1, a2, a3)
                b0, b1, b2, b3 = add_row(j + 1, b0, b1, b2, b3)
                return a0, a1, a2, a3, b0, b1, b2, b3

            def accum1(j, carry):
                a0, a1, a2, a3, b0, b1, b2, b3 = carry
                a0, a1, a2, a3 = add_row(j, a0, a1, a2, a3)
                return a0, a1, a2, a3, b0, b1, b2, b3

            carry = (z, z, z, z, z, z, z, z)
            carry = pl.loop(0, n2, step=2, init_carry=carry)(accum2)
            carry = pl.loop(n2, n, init_carry=carry)(accum1)
            a0, a1, a2, a3, b0, b1, b2, b3 = carry
            acc[s, pl.ds(0, 16)] = a0 + b0
            acc[s, pl.ds(16, 16)] = a1 + b1
            acc[s, pl.ds(32, 16)] = a2 + b2
            acc[s, pl.ds(48, 16)] = a3 + b3

        fire(0, buf0, sem0)

        @pl.loop(0, _SPT, step=2)
        def _(s0):
            fire(s0 + 1, buf1, sem1)
            drain(s0, buf0, sem0)
            process(s0, buf0)
            s2 = jnp.where(s0 + 2 >= _SPT, 0, s0 + 2)
            fire(s2, buf0, sem0)
            drain(s0 + 1, buf1, sem1)
            process(s0 + 1, buf1)

        drain(0, buf0, sem0)
        pltpu.sync_copy(acc, sums_hbm.at[pl.ds(base, _SPT)])

    run = pl.kernel(
        body,
        out_type=jax.ShapeDtypeStruct((_B, _DP), jnp.float32),
        mesh=mesh,
        scratch_types=[
            pltpu.VMEM((_SPT, _L), jnp.int32),
            pltpu.VMEM((_SPT + 16,), jnp.int32),
            pltpu.VMEM((_SPT, _DP), jnp.float32),
            pltpu.VMEM((_LP, _W32), jnp.int32),
            pltpu.VMEM((_LP, _W32), jnp.int32),
            pltpu.SemaphoreType.DMA,
            pltpu.SemaphoreType.DMA,
        ],
        compiler_params=pltpu.CompilerParams(use_tc_tiling_on_sc=False,
                                             needs_layout_passes=False),
    )
    return run(x, lengths, packed)


def _head_body(sums_ref, len_ref, w_ref, b_ref, out_ref):
    s = sums_ref[:, :_D]
    l = len_ref[...].astype(jnp.float32)
    rep = jnp.maximum(s / l, 0.0)
    out_ref[...] = lax.dot_general(
        rep, w_ref[...], (((1,), (1,)), ((), ())),
        preferred_element_type=jnp.float32) + b_ref[...]


def _head(sums, lengths, W, b):
    return pl.pallas_call(
        _head_body,
        out_shape=jax.ShapeDtypeStruct((_B, _OUT), jnp.float32),
    )(sums, lengths, W, b)


def kernel(x, lengths, table, W, b):
    xi = x.astype(jnp.int32)
    xi = 4 * (xi % (_V // 4)) + xi // (_V // 4)
    li = lengths.astype(jnp.int32)
    packed = _pack(table).reshape(_V, _W32)
    sums = _embed_sums(xi, li, packed)
    return _head(sums, li.reshape(_B, 1), W, b.reshape(1, _OUT))
